# baseline (speedup 1.0000x reference)
import functools

import jax
import jax.numpy as jnp
from jax import lax
from jax.experimental import pallas as pl
from jax.experimental.pallas import tpu as pltpu
from jax.experimental.pallas import tpu_sc as plsc

N = 10000
E = 65536
D_IN = 1024
D_HID = 256
HALF = 128
D_OUT = 1024
NG = 64

NC = 2
NS = 16
TROWS = 640
TROWS_LAST = N - (NS - 1) * TROWS
K = 128
CH_AGG = E // NS // K
CH_DEG = E // (NC * NS) // K

_mesh = plsc.VectorSubcoreMesh(core_axis_name="c", subcore_axis_name="s")


def _tile_copy(sid, mk_src, mk_dst):
    @pl.when(sid < NS - 1)
    def _main():
        r0 = pl.multiple_of(sid * TROWS, 8)
        pltpu.sync_copy(mk_src(r0, TROWS), mk_dst(r0, TROWS))

    @pl.when(sid == NS - 1)
    def _last():
        r0 = (NS - 1) * TROWS
        pltpu.sync_copy(mk_src(r0, TROWS_LAST), mk_dst(r0, TROWS_LAST))


def _deg_body(dst4, ones_hbm, z16, out, idx_d, ones_v, sem, acc):
    cid = lax.axis_index("c")
    sid = lax.axis_index("s")
    _tile_copy(sid,
               lambda r0, n: z16.at[pl.ds(0, n)],
               lambda r0, n: acc.at[pl.ds(r0, n)])
    pltpu.sync_copy(dst4.at[cid, sid], idx_d)
    pltpu.sync_copy(ones_hbm, ones_v)
    plsc.subcore_barrier()

    def chunk(g, carry):
        pltpu.sync_copy(ones_v, acc.at[idx_d.at[g]], add=True)
        return carry

    lax.fori_loop(0, CH_DEG, chunk, 0)
    plsc.subcore_barrier()
    _tile_copy(sid,
               lambda r0, n: acc.at[pl.ds(r0, n)],
               lambda r0, n: out.at[cid, pl.ds(r0, n)])


_deg_call = pl.kernel(
    _deg_body,
    out_type=jax.ShapeDtypeStruct((NC, N, 16), jnp.float32),
    mesh=_mesh,
    compiler_params=pltpu.CompilerParams(use_tc_tiling_on_sc=False),
    scratch_types=[
        pltpu.VMEM((CH_DEG, K), jnp.int32),
        pltpu.VMEM((K, 16), jnp.float32),
        pltpu.SemaphoreType.DMA,
        pltpu.VMEM_SHARED((N, 16), jnp.float32),
    ],
)


_NBUF = 2


def _agg_body(ut, src2, dstw, z128, out, idx_s, idx_d, m0, m1,
              zsem, gsems, acc):
    msgs = [m0, m1]
    cid = lax.axis_index("c")
    sid = lax.axis_index("s")
    pltpu.sync_copy(src2.at[cid, sid], idx_s)
    pltpu.sync_copy(dstw.at[sid], idx_d)
    @pl.when(sid < NS - 1)
    def _zmain():
        r0 = pl.multiple_of(sid * TROWS, 8)
        pltpu.async_copy(z128.at[pl.ds(0, TROWS)],
                         acc.at[pl.ds(r0, TROWS)], zsem)

    @pl.when(sid == NS - 1)
    def _zlast():
        r0 = (NS - 1) * TROWS
        pltpu.async_copy(z128.at[pl.ds(0, TROWS_LAST)],
                         acc.at[pl.ds(r0, TROWS_LAST)], zsem)

    for b in range(_NBUF):
        pltpu.async_copy(ut.at[idx_s.at[b]], msgs[b], gsems[b])

    @pl.when(sid < NS - 1)
    def _zwmain():
        r0 = pl.multiple_of(sid * TROWS, 8)
        pltpu.make_async_copy(z128.at[pl.ds(0, TROWS)],
                              acc.at[pl.ds(r0, TROWS)], zsem).wait()

    @pl.when(sid == NS - 1)
    def _zwlast():
        r0 = (NS - 1) * TROWS
        pltpu.make_async_copy(z128.at[pl.ds(0, TROWS_LAST)],
                              acc.at[pl.ds(r0, TROWS_LAST)], zsem).wait()

    plsc.subcore_barrier()

    def pair(j, carry):
        c0 = _NBUF * j
        for b in range(_NBUF):
            c = c0 + b
            pltpu.make_async_copy(ut.at[idx_s.at[c]], msgs[b],
                                  gsems[b]).wait()
            pltpu.sync_copy(msgs[b], acc.at[idx_d.at[c]], add=True)
            nxt = lax.rem(c + _NBUF, CH_AGG)
            pltpu.async_copy(ut.at[idx_s.at[nxt]], msgs[b], gsems[b])
        return carry

    lax.fori_loop(0, CH_AGG // _NBUF, pair, 0)
    for b in range(_NBUF):
        pltpu.make_async_copy(ut.at[idx_s.at[b]], msgs[b],
                              gsems[b]).wait()
    plsc.subcore_barrier()
    _tile_copy(sid,
               lambda r0, n: acc.at[pl.ds(r0, n)],
               lambda r0, n: out.at[pl.ds(pl.multiple_of(cid * N + r0, 8), n)])


_agg_call = pl.kernel(
    _agg_body,
    out_type=jax.ShapeDtypeStruct((NC * N, HALF), jnp.float32),
    mesh=_mesh,
    scratch_types=[
        pltpu.VMEM((CH_AGG, K), jnp.int32),
        pltpu.VMEM((CH_AGG, K), jnp.int32),
        pltpu.VMEM((K, HALF), jnp.float32),
        pltpu.VMEM((K, HALF), jnp.float32),
        pltpu.SemaphoreType.DMA,
        [pltpu.SemaphoreType.DMA] * _NBUF,
        pltpu.VMEM_SHARED((N, HALF), jnp.float32),
    ],
)


_RB = 1000


def _fused1_body(degp_ref, x_ref, w_ref, dinv_ref, ut_ref):
    deg = degp_ref[0] + degp_ref[1] + 1.0
    dinv = lax.rsqrt(deg)
    dinv_ref[...] = dinv
    xw = jnp.dot(x_ref[...], w_ref[...],
                 preferred_element_type=jnp.float32)
    u = xw * dinv[:, :1]
    ut_ref[0] = u[:, :HALF]
    ut_ref[1] = u[:, HALF:]


def _fused1_call(degp, x, w):
    return pl.pallas_call(
        _fused1_body,
        grid=(N // _RB,),
        in_specs=[
            pl.BlockSpec((NC, _RB, 16), lambda i: (0, i, 0)),
            pl.BlockSpec((_RB, D_IN), lambda i: (i, 0)),
            pl.BlockSpec((D_IN, D_HID), lambda i: (0, 0)),
        ],
        out_specs=[
            pl.BlockSpec((_RB, 16), lambda i: (i, 0)),
            pl.BlockSpec((NC, _RB, HALF), lambda i: (0, i, 0)),
        ],
        out_shape=[
            jax.ShapeDtypeStruct((N, 16), jnp.float32),
            jax.ShapeDtypeStruct((NC, N, HALF), jnp.float32),
        ],
    )(degp, x, w)


def _layer1_body(s1_ref, ut1_ref, dinv_ref, b1_ref, ut2_ref):
    d1 = dinv_ref[:, :1]
    s = jnp.concatenate([s1_ref[0], s1_ref[1]], axis=1)
    u1 = jnp.concatenate([ut1_ref[0], ut1_ref[1]], axis=1)
    h = d1 * (s + u1) + b1_ref[...]
    h = jnp.maximum(h, 0.0)
    u2 = d1 * h
    ut2_ref[0] = u2[:, :HALF]
    ut2_ref[1] = u2[:, HALF:]


def _layer1_call(s1, ut1, dinv, b1r):
    return pl.pallas_call(
        _layer1_body,
        grid=(N // _RB,),
        in_specs=[
            pl.BlockSpec((NC, _RB, HALF), lambda i: (0, i, 0)),
            pl.BlockSpec((NC, _RB, HALF), lambda i: (0, i, 0)),
            pl.BlockSpec((_RB, 16), lambda i: (i, 0)),
            pl.BlockSpec((1, D_HID), lambda i: (0, 0)),
        ],
        out_specs=pl.BlockSpec((NC, _RB, HALF), lambda i: (0, i, 0)),
        out_shape=jax.ShapeDtypeStruct((NC, N, HALF), jnp.float32),
    )(s1, ut1, dinv, b1r)


def _pool_body(s2_ref, ut2_ref, dinv_ref, batch_ref, w2_ref, b2_ref,
               out_ref, acc_ref, cnt_ref):
    i = pl.program_id(0)

    @pl.when(i == 0)
    def _init():
        acc_ref[...] = jnp.zeros_like(acc_ref)
        cnt_ref[...] = jnp.zeros_like(cnt_ref)

    d1 = dinv_ref[:, :1]
    s = jnp.concatenate([s2_ref[0], s2_ref[1]], axis=1)
    u2 = jnp.concatenate([ut2_ref[0], ut2_ref[1]], axis=1)
    v = d1 * (s + u2)
    b = batch_ref[0, 0, :]
    onehot_t = (lax.broadcasted_iota(jnp.int32, (NG, _RB), 0)
                == b[None, :]).astype(jnp.float32)
    acc_ref[...] += jnp.dot(onehot_t, v, preferred_element_type=jnp.float32)
    cnt_ref[...] += jnp.sum(onehot_t, axis=1, keepdims=True)

    @pl.when(i == (N // _RB) - 1)
    def _fin():
        cnt = cnt_ref[:, :1]
        mean = acc_ref[...] / jnp.maximum(cnt, 1.0)
        o = jnp.dot(mean, w2_ref[...],
                    preferred_element_type=jnp.float32) + b2_ref[...]
        out_ref[...] = jnp.where(cnt > 0.0, o, 0.0)


def _pool_call(s2, ut2, dinv, batchr, w2, b2r):
    return pl.pallas_call(
        _pool_body,
        grid=(N // _RB,),
        in_specs=[
            pl.BlockSpec((NC, _RB, HALF), lambda i: (0, i, 0)),
            pl.BlockSpec((NC, _RB, HALF), lambda i: (0, i, 0)),
            pl.BlockSpec((_RB, 16), lambda i: (i, 0)),
            pl.BlockSpec((1, 1, _RB), lambda i: (i, 0, 0)),
            pl.BlockSpec((D_HID, D_OUT), lambda i: (0, 0)),
            pl.BlockSpec((1, D_OUT), lambda i: (0, 0)),
        ],
        out_specs=pl.BlockSpec((NG, D_OUT), lambda i: (0, 0)),
        out_shape=jax.ShapeDtypeStruct((NG, D_OUT), jnp.float32),
        scratch_shapes=[
            pltpu.VMEM((NG, D_HID), jnp.float32),
            pltpu.VMEM((NG, HALF), jnp.float32),
        ],
    )(s2, ut2, dinv, batchr, w2, b2r)


@jax.jit
def kernel(x, edge_index, batch, W1, b1, W2, b2):
    src = edge_index[0]
    dst = edge_index[1]
    src2 = jnp.stack([src, src + N]).reshape(NC, NS, CH_AGG, K)
    dstw = dst.reshape(NS, CH_AGG, K)
    dst4 = dst.reshape(NC, NS, CH_DEG, K)
    ones16 = jnp.ones((K, 16), jnp.float32)
    z16 = jnp.zeros((TROWS, 16), jnp.float32)
    z128 = jnp.zeros((TROWS, HALF), jnp.float32)

    degp = _deg_call(dst4, ones16, z16)
    dinv, ut1 = _fused1_call(degp, x, W1)
    s1 = _agg_call(ut1.reshape(NC * N, HALF), src2, dstw, z128)
    ut2 = _layer1_call(s1.reshape(NC, N, HALF), ut1, dinv,
                       b1.reshape(1, D_HID))
    s2 = _agg_call(ut2.reshape(NC * N, HALF), src2, dstw, z128)
    return _pool_call(s2.reshape(NC, N, HALF), ut2, dinv,
                      batch.reshape(N // _RB, 1, _RB), W2,
                      b2.reshape(1, D_OUT))


# speedup vs baseline: 1.0024x; 1.0024x over previous
import functools

import jax
import jax.numpy as jnp
from jax import lax
from jax.experimental import pallas as pl
from jax.experimental.pallas import tpu as pltpu
from jax.experimental.pallas import tpu_sc as plsc

N = 10000
E = 65536
D_IN = 1024
D_HID = 256
HALF = 128
D_OUT = 1024
NG = 64

NC = 2
NS = 16
TROWS = 640
TROWS_LAST = N - (NS - 1) * TROWS
K = 128
CH_AGG = E // NS // K
CH_DEG = E // (NC * NS) // K

_mesh = plsc.VectorSubcoreMesh(core_axis_name="c", subcore_axis_name="s")


def _tile_copy(sid, mk_src, mk_dst):
    @pl.when(sid < NS - 1)
    def _main():
        r0 = pl.multiple_of(sid * TROWS, 8)
        pltpu.sync_copy(mk_src(r0, TROWS), mk_dst(r0, TROWS))

    @pl.when(sid == NS - 1)
    def _last():
        r0 = (NS - 1) * TROWS
        pltpu.sync_copy(mk_src(r0, TROWS_LAST), mk_dst(r0, TROWS_LAST))


def _deg_body(dst4, ones_hbm, z16, out, idx_d, ones_v, sem, acc):
    cid = lax.axis_index("c")
    sid = lax.axis_index("s")
    _tile_copy(sid,
               lambda r0, n: z16.at[pl.ds(0, n)],
               lambda r0, n: acc.at[pl.ds(r0, n)])
    pltpu.sync_copy(dst4.at[cid, sid], idx_d)
    pltpu.sync_copy(ones_hbm, ones_v)
    plsc.subcore_barrier()

    def chunk(g, carry):
        pltpu.sync_copy(ones_v, acc.at[idx_d.at[g]], add=True)
        return carry

    lax.fori_loop(0, CH_DEG, chunk, 0)
    plsc.subcore_barrier()
    _tile_copy(sid,
               lambda r0, n: acc.at[pl.ds(r0, n)],
               lambda r0, n: out.at[cid, pl.ds(r0, n)])


_deg_call = pl.kernel(
    _deg_body,
    out_type=jax.ShapeDtypeStruct((NC, N, 16), jnp.float32),
    mesh=_mesh,
    compiler_params=pltpu.CompilerParams(use_tc_tiling_on_sc=False),
    scratch_types=[
        pltpu.VMEM((CH_DEG, K), jnp.int32),
        pltpu.VMEM((K, 16), jnp.float32),
        pltpu.SemaphoreType.DMA,
        pltpu.VMEM_SHARED((N, 16), jnp.float32),
    ],
)


_NBUF = 2


def _agg_body(ut, src2, dstw, z128, out, idx_s, idx_d, m0, m1,
              zsem, gsems, acc):
    msgs = [m0, m1]
    cid = lax.axis_index("c")
    sid = lax.axis_index("s")
    pltpu.sync_copy(src2.at[cid, sid], idx_s)
    pltpu.sync_copy(dstw.at[sid], idx_d)
    @pl.when(sid < NS - 1)
    def _zmain():
        r0 = pl.multiple_of(sid * TROWS, 8)
        pltpu.async_copy(z128.at[pl.ds(0, TROWS)],
                         acc.at[pl.ds(r0, TROWS)], zsem)

    @pl.when(sid == NS - 1)
    def _zlast():
        r0 = (NS - 1) * TROWS
        pltpu.async_copy(z128.at[pl.ds(0, TROWS_LAST)],
                         acc.at[pl.ds(r0, TROWS_LAST)], zsem)

    for b in range(_NBUF):
        pltpu.async_copy(ut.at[idx_s.at[b]], msgs[b], gsems[b])

    @pl.when(sid < NS - 1)
    def _zwmain():
        r0 = pl.multiple_of(sid * TROWS, 8)
        pltpu.make_async_copy(z128.at[pl.ds(0, TROWS)],
                              acc.at[pl.ds(r0, TROWS)], zsem).wait()

    @pl.when(sid == NS - 1)
    def _zwlast():
        r0 = (NS - 1) * TROWS
        pltpu.make_async_copy(z128.at[pl.ds(0, TROWS_LAST)],
                              acc.at[pl.ds(r0, TROWS_LAST)], zsem).wait()

    plsc.subcore_barrier()

    def pair(j, carry):
        c0 = _NBUF * j
        for b in range(_NBUF):
            c = c0 + b
            pltpu.make_async_copy(ut.at[idx_s.at[c]], msgs[b],
                                  gsems[b]).wait()
            pltpu.sync_copy(msgs[b], acc.at[idx_d.at[c]], add=True)
            nxt = lax.rem(c + _NBUF, CH_AGG)
            pltpu.async_copy(ut.at[idx_s.at[nxt]], msgs[b], gsems[b])
        return carry

    lax.fori_loop(0, CH_AGG // _NBUF, pair, 0)
    for b in range(_NBUF):
        pltpu.make_async_copy(ut.at[idx_s.at[b]], msgs[b],
                              gsems[b]).wait()
    plsc.subcore_barrier()
    _tile_copy(sid,
               lambda r0, n: acc.at[pl.ds(r0, n)],
               lambda r0, n: out.at[pl.ds(pl.multiple_of(cid * N + r0, 8), n)])


_agg_call = pl.kernel(
    _agg_body,
    out_type=jax.ShapeDtypeStruct((NC * N, HALF), jnp.float32),
    mesh=_mesh,
    scratch_types=[
        pltpu.VMEM((CH_AGG, K), jnp.int32),
        pltpu.VMEM((CH_AGG, K), jnp.int32),
        pltpu.VMEM((K, HALF), jnp.float32),
        pltpu.VMEM((K, HALF), jnp.float32),
        pltpu.SemaphoreType.DMA,
        [pltpu.SemaphoreType.DMA] * _NBUF,
        pltpu.VMEM_SHARED((N, HALF), jnp.float32),
    ],
)


_RB = 1000


def _fused1_body(degp_ref, x_ref, w_ref, dinv_ref, ut_ref):
    deg = degp_ref[0] + degp_ref[1] + 1.0
    dinv = lax.rsqrt(deg)
    dinv_ref[...] = dinv
    xw = jnp.dot(x_ref[...].astype(jnp.bfloat16),
                 w_ref[...].astype(jnp.bfloat16),
                 preferred_element_type=jnp.float32)
    u = xw * dinv[:, :1]
    ut_ref[0] = u[:, :HALF]
    ut_ref[1] = u[:, HALF:]


def _fused1_call(degp, x, w):
    return pl.pallas_call(
        _fused1_body,
        grid=(N // _RB,),
        in_specs=[
            pl.BlockSpec((NC, _RB, 16), lambda i: (0, i, 0)),
            pl.BlockSpec((_RB, D_IN), lambda i: (i, 0)),
            pl.BlockSpec((D_IN, D_HID), lambda i: (0, 0)),
        ],
        out_specs=[
            pl.BlockSpec((_RB, 16), lambda i: (i, 0)),
            pl.BlockSpec((NC, _RB, HALF), lambda i: (0, i, 0)),
        ],
        out_shape=[
            jax.ShapeDtypeStruct((N, 16), jnp.float32),
            jax.ShapeDtypeStruct((NC, N, HALF), jnp.float32),
        ],
    )(degp, x, w)


def _layer1_body(s1_ref, ut1_ref, dinv_ref, b1_ref, ut2_ref):
    d1 = dinv_ref[:, :1]
    for c in range(NC):
        h = d1 * (s1_ref[c] + ut1_ref[c]) + b1_ref[:, pl.ds(c * HALF, HALF)]
        ut2_ref[c] = d1 * jnp.maximum(h, 0.0)


def _layer1_call(s1, ut1, dinv, b1r):
    return pl.pallas_call(
        _layer1_body,
        grid=(N // _RB,),
        in_specs=[
            pl.BlockSpec((NC, _RB, HALF), lambda i: (0, i, 0)),
            pl.BlockSpec((NC, _RB, HALF), lambda i: (0, i, 0)),
            pl.BlockSpec((_RB, 16), lambda i: (i, 0)),
            pl.BlockSpec((1, D_HID), lambda i: (0, 0)),
        ],
        out_specs=pl.BlockSpec((NC, _RB, HALF), lambda i: (0, i, 0)),
        out_shape=jax.ShapeDtypeStruct((NC, N, HALF), jnp.float32),
    )(s1, ut1, dinv, b1r)


def _pool_body(s2_ref, ut2_ref, dinv_ref, batch_ref, w2_ref, b2_ref,
               out_ref, acc_ref, cnt_ref):
    i = pl.program_id(0)

    @pl.when(i == 0)
    def _init():
        acc_ref[...] = jnp.zeros_like(acc_ref)
        cnt_ref[...] = jnp.zeros_like(cnt_ref)

    d1 = dinv_ref[:, :1]
    b = batch_ref[0, 0, :]
    onehot_t = (lax.broadcasted_iota(jnp.int32, (NG, _RB), 0)
                == b[None, :]).astype(jnp.float32)
    for c in range(NC):
        v = d1 * (s2_ref[c] + ut2_ref[c])
        acc_ref[:, pl.ds(c * HALF, HALF)] += jnp.dot(
            onehot_t, v, preferred_element_type=jnp.float32)
    cnt_ref[...] += jnp.sum(onehot_t, axis=1, keepdims=True)

    @pl.when(i == (N // _RB) - 1)
    def _fin():
        cnt = cnt_ref[:, :1]
        mean = acc_ref[...] / jnp.maximum(cnt, 1.0)
        o = jnp.dot(mean, w2_ref[...],
                    preferred_element_type=jnp.float32) + b2_ref[...]
        out_ref[...] = jnp.where(cnt > 0.0, o, 0.0)


def _pool_call(s2, ut2, dinv, batchr, w2, b2r):
    return pl.pallas_call(
        _pool_body,
        grid=(N // _RB,),
        in_specs=[
            pl.BlockSpec((NC, _RB, HALF), lambda i: (0, i, 0)),
            pl.BlockSpec((NC, _RB, HALF), lambda i: (0, i, 0)),
            pl.BlockSpec((_RB, 16), lambda i: (i, 0)),
            pl.BlockSpec((1, 1, _RB), lambda i: (i, 0, 0)),
            pl.BlockSpec((D_HID, D_OUT), lambda i: (0, 0)),
            pl.BlockSpec((1, D_OUT), lambda i: (0, 0)),
        ],
        out_specs=pl.BlockSpec((NG, D_OUT), lambda i: (0, 0)),
        out_shape=jax.ShapeDtypeStruct((NG, D_OUT), jnp.float32),
        scratch_shapes=[
            pltpu.VMEM((NG, D_HID), jnp.float32),
            pltpu.VMEM((NG, HALF), jnp.float32),
        ],
    )(s2, ut2, dinv, batchr, w2, b2r)


@jax.jit
def kernel(x, edge_index, batch, W1, b1, W2, b2):
    src = edge_index[0]
    dst = edge_index[1]
    src2 = jnp.stack([src, src + N]).reshape(NC, NS, CH_AGG, K)
    dstw = dst.reshape(NS, CH_AGG, K)
    dst4 = dst.reshape(NC, NS, CH_DEG, K)
    ones16 = jnp.ones((K, 16), jnp.float32)
    z16 = jnp.zeros((TROWS, 16), jnp.float32)
    z128 = jnp.zeros((TROWS, HALF), jnp.float32)

    degp = _deg_call(dst4, ones16, z16)
    dinv, ut1 = _fused1_call(degp, x, W1)
    s1 = _agg_call(ut1.reshape(NC * N, HALF), src2, dstw, z128)
    ut2 = _layer1_call(s1.reshape(NC, N, HALF), ut1, dinv,
                       b1.reshape(1, D_HID))
    s2 = _agg_call(ut2.reshape(NC * N, HALF), src2, dstw, z128)
    return _pool_call(s2.reshape(NC, N, HALF), ut2, dinv,
                      batch.reshape(N // _RB, 1, _RB), W2,
                      b2.reshape(1, D_OUT))


# speedup vs baseline: 1.0450x; 1.0425x over previous
import functools

import jax
import jax.numpy as jnp
from jax import lax
from jax.experimental import pallas as pl
from jax.experimental.pallas import tpu as pltpu
from jax.experimental.pallas import tpu_sc as plsc

N = 10000
E = 65536
D_IN = 1024
D_HID = 256
HALF = 128
D_OUT = 1024
NG = 64

NC = 2
NS = 16
TROWS = 640
TROWS_LAST = N - (NS - 1) * TROWS
K = 128
CH_AGG = E // NS // K
CH_DEG = E // (NC * NS) // K

_mesh = plsc.VectorSubcoreMesh(core_axis_name="c", subcore_axis_name="s")


def _tile_copy(sid, mk_src, mk_dst):
    @pl.when(sid < NS - 1)
    def _main():
        r0 = pl.multiple_of(sid * TROWS, 8)
        pltpu.sync_copy(mk_src(r0, TROWS), mk_dst(r0, TROWS))

    @pl.when(sid == NS - 1)
    def _last():
        r0 = (NS - 1) * TROWS
        pltpu.sync_copy(mk_src(r0, TROWS_LAST), mk_dst(r0, TROWS_LAST))


def _deg_body(dst4, ones_hbm, z16, out, idx_d, ones_v, sem, acc):
    cid = lax.axis_index("c")
    sid = lax.axis_index("s")
    _tile_copy(sid,
               lambda r0, n: z16.at[pl.ds(0, n)],
               lambda r0, n: acc.at[pl.ds(r0, n)])
    pltpu.sync_copy(dst4.at[cid, sid], idx_d)
    pltpu.sync_copy(ones_hbm, ones_v)
    plsc.subcore_barrier()

    def chunk(g, carry):
        pltpu.sync_copy(ones_v, acc.at[idx_d.at[g]], add=True)
        return carry

    lax.fori_loop(0, CH_DEG, chunk, 0)
    plsc.subcore_barrier()
    _tile_copy(sid,
               lambda r0, n: acc.at[pl.ds(r0, n)],
               lambda r0, n: out.at[cid, pl.ds(r0, n)])


_deg_call = pl.kernel(
    _deg_body,
    out_type=jax.ShapeDtypeStruct((NC, N, 16), jnp.float32),
    mesh=_mesh,
    compiler_params=pltpu.CompilerParams(use_tc_tiling_on_sc=False),
    scratch_types=[
        pltpu.VMEM((CH_DEG, K), jnp.int32),
        pltpu.VMEM((K, 16), jnp.float32),
        pltpu.SemaphoreType.DMA,
        pltpu.VMEM_SHARED((N, 16), jnp.float32),
    ],
)


_NBUF = 2


def _agg_body(ut, src2, dstw, out, idx_s, idx_d, m0, m1,
              zsem, gsems, acc):
    msgs = [m0, m1]
    cid = lax.axis_index("c")
    sid = lax.axis_index("s")
    pltpu.sync_copy(src2.at[cid, sid], idx_s)
    pltpu.sync_copy(dstw.at[sid], idx_d)
    @pl.when(sid < NS - 1)
    def _zmain():
        r0 = pl.multiple_of(sid * TROWS, 8)
        pltpu.async_copy(ut.at[pl.ds(pl.multiple_of(cid * N + r0, 8), TROWS)],
                         acc.at[pl.ds(r0, TROWS)], zsem)

    @pl.when(sid == NS - 1)
    def _zlast():
        r0 = (NS - 1) * TROWS
        pltpu.async_copy(ut.at[pl.ds(cid * N + r0, TROWS_LAST)],
                         acc.at[pl.ds(r0, TROWS_LAST)], zsem)

    for b in range(_NBUF):
        pltpu.async_copy(ut.at[idx_s.at[b]], msgs[b], gsems[b])

    @pl.when(sid < NS - 1)
    def _zwmain():
        r0 = pl.multiple_of(sid * TROWS, 8)
        pltpu.make_async_copy(ut.at[pl.ds(pl.multiple_of(cid * N + r0, 8),
                                          TROWS)],
                              acc.at[pl.ds(r0, TROWS)], zsem).wait()

    @pl.when(sid == NS - 1)
    def _zwlast():
        r0 = (NS - 1) * TROWS
        pltpu.make_async_copy(ut.at[pl.ds(cid * N + r0, TROWS_LAST)],
                              acc.at[pl.ds(r0, TROWS_LAST)], zsem).wait()

    plsc.subcore_barrier()

    def pair(j, carry):
        c0 = _NBUF * j
        for b in range(_NBUF):
            c = c0 + b
            pltpu.make_async_copy(ut.at[idx_s.at[c]], msgs[b],
                                  gsems[b]).wait()
            pltpu.sync_copy(msgs[b], acc.at[idx_d.at[c]], add=True)
            nxt = lax.rem(c + _NBUF, CH_AGG)
            pltpu.async_copy(ut.at[idx_s.at[nxt]], msgs[b], gsems[b])
        return carry

    lax.fori_loop(0, CH_AGG // _NBUF, pair, 0)
    for b in range(_NBUF):
        pltpu.make_async_copy(ut.at[idx_s.at[b]], msgs[b],
                              gsems[b]).wait()
    plsc.subcore_barrier()
    _tile_copy(sid,
               lambda r0, n: acc.at[pl.ds(r0, n)],
               lambda r0, n: out.at[pl.ds(pl.multiple_of(cid * N + r0, 8), n)])


_agg_call = pl.kernel(
    _agg_body,
    out_type=jax.ShapeDtypeStruct((NC * N, HALF), jnp.float32),
    mesh=_mesh,
    scratch_types=[
        pltpu.VMEM((CH_AGG, K), jnp.int32),
        pltpu.VMEM((CH_AGG, K), jnp.int32),
        pltpu.VMEM((K, HALF), jnp.float32),
        pltpu.VMEM((K, HALF), jnp.float32),
        pltpu.SemaphoreType.DMA,
        [pltpu.SemaphoreType.DMA] * _NBUF,
        pltpu.VMEM_SHARED((N, HALF), jnp.float32),
    ],
)


_RB = 1000


def _fused1_body(degp_ref, x_ref, w_ref, dinv_ref, ut_ref):
    deg = degp_ref[0] + degp_ref[1] + 1.0
    dinv = lax.rsqrt(deg)
    dinv_ref[...] = dinv
    xw = jnp.dot(x_ref[...].astype(jnp.bfloat16),
                 w_ref[...].astype(jnp.bfloat16),
                 preferred_element_type=jnp.float32)
    u = xw * dinv[:, :1]
    ut_ref[0] = u[:, :HALF]
    ut_ref[1] = u[:, HALF:]


def _fused1_call(degp, x, w):
    return pl.pallas_call(
        _fused1_body,
        grid=(N // _RB,),
        in_specs=[
            pl.BlockSpec((NC, _RB, 16), lambda i: (0, i, 0)),
            pl.BlockSpec((_RB, D_IN), lambda i: (i, 0)),
            pl.BlockSpec((D_IN, D_HID), lambda i: (0, 0)),
        ],
        out_specs=[
            pl.BlockSpec((_RB, 16), lambda i: (i, 0)),
            pl.BlockSpec((NC, _RB, HALF), lambda i: (0, i, 0)),
        ],
        out_shape=[
            jax.ShapeDtypeStruct((N, 16), jnp.float32),
            jax.ShapeDtypeStruct((NC, N, HALF), jnp.float32),
        ],
    )(degp, x, w)


def _layer1_body(s1_ref, dinv_ref, b1_ref, ut2_ref):
    d1 = dinv_ref[:, :1]
    for c in range(NC):
        h = d1 * s1_ref[c] + b1_ref[:, pl.ds(c * HALF, HALF)]
        ut2_ref[c] = d1 * jnp.maximum(h, 0.0)


def _layer1_call(s1, dinv, b1r):
    return pl.pallas_call(
        _layer1_body,
        grid=(N // _RB,),
        in_specs=[
            pl.BlockSpec((NC, _RB, HALF), lambda i: (0, i, 0)),
            pl.BlockSpec((_RB, 16), lambda i: (i, 0)),
            pl.BlockSpec((1, D_HID), lambda i: (0, 0)),
        ],
        out_specs=pl.BlockSpec((NC, _RB, HALF), lambda i: (0, i, 0)),
        out_shape=jax.ShapeDtypeStruct((NC, N, HALF), jnp.float32),
    )(s1, dinv, b1r)


def _pool_body(s2_ref, dinv_ref, batch_ref, w2_ref, b2_ref,
               out_ref, acc_ref, cnt_ref):
    i = pl.program_id(0)

    @pl.when(i == 0)
    def _init():
        acc_ref[...] = jnp.zeros_like(acc_ref)
        cnt_ref[...] = jnp.zeros_like(cnt_ref)

    d1 = dinv_ref[:, :1]
    b = batch_ref[0, 0, :]
    onehot_t = (lax.broadcasted_iota(jnp.int32, (NG, _RB), 0)
                == b[None, :]).astype(jnp.float32)
    for c in range(NC):
        v = d1 * s2_ref[c]
        acc_ref[:, pl.ds(c * HALF, HALF)] += jnp.dot(
            onehot_t, v, preferred_element_type=jnp.float32)
    cnt_ref[...] += jnp.sum(onehot_t, axis=1, keepdims=True)

    @pl.when(i == (N // _RB) - 1)
    def _fin():
        cnt = cnt_ref[:, :1]
        mean = acc_ref[...] / jnp.maximum(cnt, 1.0)
        o = jnp.dot(mean, w2_ref[...],
                    preferred_element_type=jnp.float32) + b2_ref[...]
        out_ref[...] = jnp.where(cnt > 0.0, o, 0.0)


def _pool_call(s2, dinv, batchr, w2, b2r):
    return pl.pallas_call(
        _pool_body,
        grid=(N // _RB,),
        in_specs=[
            pl.BlockSpec((NC, _RB, HALF), lambda i: (0, i, 0)),
            pl.BlockSpec((_RB, 16), lambda i: (i, 0)),
            pl.BlockSpec((1, 1, _RB), lambda i: (i, 0, 0)),
            pl.BlockSpec((D_HID, D_OUT), lambda i: (0, 0)),
            pl.BlockSpec((1, D_OUT), lambda i: (0, 0)),
        ],
        out_specs=pl.BlockSpec((NG, D_OUT), lambda i: (0, 0)),
        out_shape=jax.ShapeDtypeStruct((NG, D_OUT), jnp.float32),
        scratch_shapes=[
            pltpu.VMEM((NG, D_HID), jnp.float32),
            pltpu.VMEM((NG, HALF), jnp.float32),
        ],
    )(s2, dinv, batchr, w2, b2r)


@jax.jit
def kernel(x, edge_index, batch, W1, b1, W2, b2):
    src = edge_index[0]
    dst = edge_index[1]
    src2 = jnp.stack([src, src + N]).reshape(NC, NS, CH_AGG, K)
    dstw = dst.reshape(NS, CH_AGG, K)
    dst4 = dst.reshape(NC, NS, CH_DEG, K)
    ones16 = jnp.ones((K, 16), jnp.float32)
    z16 = jnp.zeros((TROWS, 16), jnp.float32)

    degp = _deg_call(dst4, ones16, z16)
    dinv, ut1 = _fused1_call(degp, x, W1)
    s1 = _agg_call(ut1.reshape(NC * N, HALF), src2, dstw)
    ut2 = _layer1_call(s1.reshape(NC, N, HALF), dinv,
                       b1.reshape(1, D_HID))
    s2 = _agg_call(ut2.reshape(NC * N, HALF), src2, dstw)
    return _pool_call(s2.reshape(NC, N, HALF), dinv,
                      batch.reshape(N // _RB, 1, _RB), W2,
                      b2.reshape(1, D_OUT))


# speedup vs baseline: 1.0464x; 1.0014x over previous
import jax
import jax.numpy as jnp
from jax import lax
from jax.experimental import pallas as pl
from jax.experimental.pallas import tpu as pltpu
from jax.experimental.pallas import tpu_sc as plsc

N = 10000
E = 65536
D_IN = 1024
D_HID = 256
HALF = 128
D_OUT = 1024
NG = 64

NC = 2
NS = 16
TROWS = 640
TROWS_LAST = N - (NS - 1) * TROWS
K = 128
CH_AGG = E // NS // K
CH_DEG = E // (NC * NS) // K

_mesh = plsc.VectorSubcoreMesh(core_axis_name="c", subcore_axis_name="s")


def _tile_copy(sid, mk_src, mk_dst):
    @pl.when(sid < NS - 1)
    def _main():
        r0 = pl.multiple_of(sid * TROWS, 8)
        pltpu.sync_copy(mk_src(r0, TROWS), mk_dst(r0, TROWS))

    @pl.when(sid == NS - 1)
    def _last():
        r0 = (NS - 1) * TROWS
        pltpu.sync_copy(mk_src(r0, TROWS_LAST), mk_dst(r0, TROWS_LAST))


def _deg_body(dst4, ones_hbm, z16, out, idx_d, ones_v, sem, acc):
    cid = lax.axis_index("c")
    sid = lax.axis_index("s")
    _tile_copy(sid,
               lambda r0, n: z16.at[pl.ds(0, n)],
               lambda r0, n: acc.at[pl.ds(r0, n)])
    pltpu.sync_copy(dst4.at[cid, sid], idx_d)
    pltpu.sync_copy(ones_hbm, ones_v)
    plsc.subcore_barrier()

    def chunk(g, carry):
        pltpu.sync_copy(ones_v, acc.at[idx_d.at[g]], add=True)
        return carry

    lax.fori_loop(0, CH_DEG, chunk, 0)
    plsc.subcore_barrier()
    _tile_copy(sid,
               lambda r0, n: acc.at[pl.ds(r0, n)],
               lambda r0, n: out.at[cid, pl.ds(r0, n)])


_deg_call = pl.kernel(
    _deg_body,
    out_type=jax.ShapeDtypeStruct((NC, N, 16), jnp.float32),
    mesh=_mesh,
    compiler_params=pltpu.CompilerParams(use_tc_tiling_on_sc=False),
    scratch_types=[
        pltpu.VMEM((CH_DEG, K), jnp.int32),
        pltpu.VMEM((K, 16), jnp.float32),
        pltpu.SemaphoreType.DMA,
        pltpu.VMEM_SHARED((N, 16), jnp.float32),
    ],
)


_NBUF = 2


def _agg_body(ut, src2, dstw, out, idx_s, idx_d, m0, m1,
              zsem, gsems, acc):
    msgs = [m0, m1]
    cid = lax.axis_index("c")
    sid = lax.axis_index("s")
    pltpu.sync_copy(src2.at[cid, sid], idx_s)
    pltpu.sync_copy(dstw.at[sid], idx_d)
    @pl.when(sid < NS - 1)
    def _zmain():
        r0 = pl.multiple_of(sid * TROWS, 8)
        pltpu.async_copy(ut.at[pl.ds(pl.multiple_of(cid * N + r0, 8), TROWS)],
                         acc.at[pl.ds(r0, TROWS)], zsem)

    @pl.when(sid == NS - 1)
    def _zlast():
        r0 = (NS - 1) * TROWS
        pltpu.async_copy(ut.at[pl.ds(cid * N + r0, TROWS_LAST)],
                         acc.at[pl.ds(r0, TROWS_LAST)], zsem)

    for b in range(_NBUF):
        pltpu.async_copy(ut.at[idx_s.at[b]], msgs[b], gsems[b])

    @pl.when(sid < NS - 1)
    def _zwmain():
        r0 = pl.multiple_of(sid * TROWS, 8)
        pltpu.make_async_copy(ut.at[pl.ds(pl.multiple_of(cid * N + r0, 8),
                                          TROWS)],
                              acc.at[pl.ds(r0, TROWS)], zsem).wait()

    @pl.when(sid == NS - 1)
    def _zwlast():
        r0 = (NS - 1) * TROWS
        pltpu.make_async_copy(ut.at[pl.ds(cid * N + r0, TROWS_LAST)],
                              acc.at[pl.ds(r0, TROWS_LAST)], zsem).wait()

    plsc.subcore_barrier()

    def pair(j, carry):
        c0 = _NBUF * j
        for b in range(_NBUF):
            c = c0 + b
            pltpu.make_async_copy(ut.at[idx_s.at[c]], msgs[b],
                                  gsems[b]).wait()
            pltpu.sync_copy(msgs[b], acc.at[idx_d.at[c]], add=True)
            nxt = lax.rem(c + _NBUF, CH_AGG)
            pltpu.async_copy(ut.at[idx_s.at[nxt]], msgs[b], gsems[b])
        return carry

    lax.fori_loop(0, CH_AGG // _NBUF, pair, 0)
    for b in range(_NBUF):
        pltpu.make_async_copy(ut.at[idx_s.at[b]], msgs[b],
                              gsems[b]).wait()
    plsc.subcore_barrier()
    _tile_copy(sid,
               lambda r0, n: acc.at[pl.ds(r0, n)],
               lambda r0, n: out.at[pl.ds(pl.multiple_of(cid * N + r0, 8), n)])


_agg_call = pl.kernel(
    _agg_body,
    out_type=jax.ShapeDtypeStruct((NC * N, HALF), jnp.float32),
    mesh=_mesh,
    scratch_types=[
        pltpu.VMEM((CH_AGG, K), jnp.int32),
        pltpu.VMEM((CH_AGG, K), jnp.int32),
        pltpu.VMEM((K, HALF), jnp.float32),
        pltpu.VMEM((K, HALF), jnp.float32),
        pltpu.SemaphoreType.DMA,
        [pltpu.SemaphoreType.DMA] * _NBUF,
        pltpu.VMEM_SHARED((N, HALF), jnp.float32),
    ],
)


_RB = 1000


def _fused1_body(degp_ref, x_ref, w_ref, dinv_ref, ut_ref):
    deg = degp_ref[0] + degp_ref[1] + 1.0
    dinv = lax.rsqrt(deg)
    dinv_ref[...] = dinv
    xw = jnp.dot(x_ref[...].astype(jnp.bfloat16),
                 w_ref[...].astype(jnp.bfloat16),
                 preferred_element_type=jnp.float32)
    u = xw * dinv[:, :1]
    ut_ref[0] = u[:, :HALF]
    ut_ref[1] = u[:, HALF:]


def _fused1_call(degp, x, w):
    return pl.pallas_call(
        _fused1_body,
        grid=(N // _RB,),
        in_specs=[
            pl.BlockSpec((NC, _RB, 16), lambda i: (0, i, 0)),
            pl.BlockSpec((_RB, D_IN), lambda i: (i, 0)),
            pl.BlockSpec((D_IN, D_HID), lambda i: (0, 0)),
        ],
        out_specs=[
            pl.BlockSpec((_RB, 16), lambda i: (i, 0)),
            pl.BlockSpec((NC, _RB, HALF), lambda i: (0, i, 0)),
        ],
        out_shape=[
            jax.ShapeDtypeStruct((N, 16), jnp.float32),
            jax.ShapeDtypeStruct((NC, N, HALF), jnp.float32),
        ],
    )(degp, x, w)


def _layer1_body(s1_ref, dinv_ref, b1_ref, ut2_ref):
    d1 = dinv_ref[:, :1]
    for c in range(NC):
        h = d1 * s1_ref[c] + b1_ref[:, pl.ds(c * HALF, HALF)]
        ut2_ref[c] = d1 * jnp.maximum(h, 0.0)


def _layer1_call(s1, dinv, b1r):
    return pl.pallas_call(
        _layer1_body,
        grid=(N // _RB,),
        in_specs=[
            pl.BlockSpec((NC, _RB, HALF), lambda i: (0, i, 0)),
            pl.BlockSpec((_RB, 16), lambda i: (i, 0)),
            pl.BlockSpec((1, D_HID), lambda i: (0, 0)),
        ],
        out_specs=pl.BlockSpec((NC, _RB, HALF), lambda i: (0, i, 0)),
        out_shape=jax.ShapeDtypeStruct((NC, N, HALF), jnp.float32),
    )(s1, dinv, b1r)


def _pool_body(s2_ref, dinv_ref, batch_ref, w2_ref, b2_ref,
               out_ref, acc_ref, cnt_ref):
    i = pl.program_id(0)

    @pl.when(i == 0)
    def _init():
        acc_ref[...] = jnp.zeros_like(acc_ref)
        cnt_ref[...] = jnp.zeros_like(cnt_ref)

    d1 = dinv_ref[:, :1]
    b = batch_ref[0, 0, :]
    onehot_t = (lax.broadcasted_iota(jnp.int32, (NG, _RB), 0)
                == b[None, :]).astype(jnp.float32)
    for c in range(NC):
        v = d1 * s2_ref[c]
        acc_ref[:, pl.ds(c * HALF, HALF)] += jnp.dot(
            onehot_t, v, preferred_element_type=jnp.float32)
    cnt_ref[...] += jnp.sum(onehot_t, axis=1, keepdims=True)

    @pl.when(i == (N // _RB) - 1)
    def _fin():
        cnt = cnt_ref[:, :1]
        mean = acc_ref[...] / jnp.maximum(cnt, 1.0)
        o = jnp.dot(mean, w2_ref[...],
                    preferred_element_type=jnp.float32) + b2_ref[...]
        out_ref[...] = jnp.where(cnt > 0.0, o, 0.0)


def _pool_call(s2, dinv, batchr, w2, b2r):
    return pl.pallas_call(
        _pool_body,
        grid=(N // _RB,),
        in_specs=[
            pl.BlockSpec((NC, _RB, HALF), lambda i: (0, i, 0)),
            pl.BlockSpec((_RB, 16), lambda i: (i, 0)),
            pl.BlockSpec((1, 1, _RB), lambda i: (i, 0, 0)),
            pl.BlockSpec((D_HID, D_OUT), lambda i: (0, 0)),
            pl.BlockSpec((1, D_OUT), lambda i: (0, 0)),
        ],
        out_specs=pl.BlockSpec((NG, D_OUT), lambda i: (0, 0)),
        out_shape=jax.ShapeDtypeStruct((NG, D_OUT), jnp.float32),
        scratch_shapes=[
            pltpu.VMEM((NG, D_HID), jnp.float32),
            pltpu.VMEM((NG, HALF), jnp.float32),
        ],
    )(s2, dinv, batchr, w2, b2r)


@jax.jit
def kernel(x, edge_index, batch, W1, b1, W2, b2):
    src = edge_index[0]
    dst = edge_index[1]
    src2 = jnp.stack([src, src + N]).reshape(NC, NS, CH_AGG, K)
    dstw = dst.reshape(NS, CH_AGG, K)
    dst4 = dst.reshape(NC, NS, CH_DEG, K)
    ones16 = jnp.ones((K, 16), jnp.float32)
    z16 = jnp.zeros((TROWS, 16), jnp.float32)

    degp = _deg_call(dst4, ones16, z16)
    dinv, ut1 = _fused1_call(degp, x, W1)
    s1 = _agg_call(ut1.reshape(NC * N, HALF), src2, dstw)
    ut2 = _layer1_call(s1.reshape(NC, N, HALF), dinv,
                       b1.reshape(1, D_HID))
    s2 = _agg_call(ut2.reshape(NC * N, HALF), src2, dstw)
    return _pool_call(s2.reshape(NC, N, HALF), dinv,
                      batch.reshape(N // _RB, 1, _RB), W2,
                      b2.reshape(1, D_OUT))
